# baseline (speedup 1.0000x reference)
import functools

import jax
import jax.numpy as jnp
from jax import lax
from jax.experimental import pallas as pl
from jax.experimental.pallas import tpu as pltpu
from jax.experimental.pallas import tpu_sc as plsc

N_CORES = 2
N_SUBCORES = 16
N_WORKERS = N_CORES * N_SUBCORES
HG = 8
BLK = 128
L = 16


def _lookup_kernel(B, V, D):
  b_per_w = B // N_WORKERS
  n_hgroups = b_per_w // HG
  half = HG * BLK
  hd = D // 2
  mesh = plsc.VectorSubcoreMesh(core_axis_name="c", subcore_axis_name="s")

  @functools.partial(
      pl.kernel,
      out_type=jax.ShapeDtypeStruct((D, B), jnp.float32),
      mesh=mesh,
      scratch_types=[
          pltpu.VMEM((b_per_w,), jnp.int32),
          pltpu.VMEM((D, 2 * half), jnp.float32),
          pltpu.VMEM((hd, 2 * b_per_w), jnp.float32),
          pltpu.SemaphoreType.DMA,
      ],
      compiler_params=pltpu.CompilerParams(needs_layout_passes=False, disable_bounds_checks=True, disable_semaphore_checks=True),
  )
  def run(idx_hbm, table_hbm, out_hbm, idx_v, ring, out_stage, sem):
    wid = lax.axis_index("s") * N_CORES + lax.axis_index("c")
    base = wid * b_per_w
    pltpu.sync_copy(idx_hbm.at[pl.ds(base, b_per_w)], idx_v)
    iota = lax.iota(jnp.int32, L)
    k8 = lax.bitwise_and(iota, HG - 1)
    sel = lax.shift_right_logical(iota, 3)

    def dup_idx(g):
      return plsc.load_gather(idx_v, [g * HG + k8])

    def issue(g):
      h = (g % 2) * half
      offvec = lax.bitwise_and(dup_idx(g), -BLK)
      for k in range(HG):
        off = pl.multiple_of(offvec[k], BLK)
        pltpu.async_copy(
            table_hbm.at[:, pl.ds(off, BLK)],
            ring.at[:, pl.ds(h + k * BLK, BLK)],
            sem,
        )

    def drain(g):
      h = (g % 2) * half
      pltpu.make_async_copy(
          out_hbm.at[:, pl.ds(0, half)],
          ring.at[:, pl.ds(h, half)],
          sem,
      ).wait()

    issue(0)

    def body(g, carry):
      @pl.when(g + 1 < n_hgroups)
      def _():
        issue(g + 1)

      drain(g)
      h = (g % 2) * half
      lvec = lax.bitwise_and(dup_idx(g), BLK - 1)
      cols = h + k8 * BLK + lvec
      ocols = g * HG + k8 + sel * b_per_w
      for d in range(hd):
        rows = d + sel * hd
        vals = plsc.load_gather(ring, [rows, cols])
        plsc.store_scatter(out_stage, [jnp.full((L,), d, jnp.int32), ocols], vals)
      return carry

    lax.fori_loop(0, n_hgroups, body, 0)
    pltpu.sync_copy(
        out_stage.at[:, pl.ds(0, b_per_w)],
        out_hbm.at[pl.ds(0, hd), pl.ds(base, b_per_w)],
    )
    pltpu.sync_copy(
        out_stage.at[:, pl.ds(b_per_w, b_per_w)],
        out_hbm.at[pl.ds(hd, hd), pl.ds(base, b_per_w)],
    )

  return run


def kernel(idx, emb_weight):
  B = idx.shape[0]
  V, D = emb_weight.shape
  run = _lookup_kernel(B, V, D)
  out_t = run(idx.astype(jnp.int32), emb_weight.T)
  return out_t.T


# speedup vs baseline: 1.0051x; 1.0051x over previous
import functools

import jax
import jax.numpy as jnp
from jax import lax
from jax.experimental import pallas as pl
from jax.experimental.pallas import tpu as pltpu
from jax.experimental.pallas import tpu_sc as plsc

N_CORES = 2
N_SUBCORES = 16
N_WORKERS = N_CORES * N_SUBCORES
HG = 8
BLK = 128
L = 16


def _lookup_kernel(B, V, D):
  b_per_w = B // N_WORKERS
  n_hgroups = b_per_w // HG
  half = HG * BLK
  hd = D // 2
  mesh = plsc.VectorSubcoreMesh(core_axis_name="c", subcore_axis_name="s")

  @functools.partial(
      pl.kernel,
      out_type=jax.ShapeDtypeStruct((D, B), jnp.float32),
      mesh=mesh,
      scratch_types=[
          pltpu.VMEM((b_per_w,), jnp.int32),
          pltpu.VMEM((D, 2 * half), jnp.float32),
          pltpu.VMEM((hd, 2 * b_per_w), jnp.float32),
          pltpu.SemaphoreType.DMA,
      ],
      compiler_params=pltpu.CompilerParams(needs_layout_passes=False),
  )
  def run(idx_hbm, table_hbm, out_hbm, idx_v, ring, out_stage, sem):
    wid = lax.axis_index("s") * N_CORES + lax.axis_index("c")
    base = wid * b_per_w
    pltpu.sync_copy(idx_hbm.at[pl.ds(base, b_per_w)], idx_v)
    iota = lax.iota(jnp.int32, L)
    k8 = lax.bitwise_and(iota, HG - 1)
    sel = lax.shift_right_logical(iota, 3)

    def dup_idx(g):
      return plsc.load_gather(idx_v, [g * HG + k8])

    def issue(g):
      h = (g % 2) * half
      offvec = lax.bitwise_and(dup_idx(g), -BLK)
      for k in range(HG):
        off = pl.multiple_of(offvec[k], BLK)
        pltpu.async_copy(
            table_hbm.at[:, pl.ds(off, BLK)],
            ring.at[:, pl.ds(h + k * BLK, BLK)],
            sem,
        )

    def drain(g):
      h = (g % 2) * half
      pltpu.make_async_copy(
          out_hbm.at[:, pl.ds(0, half)],
          ring.at[:, pl.ds(h, half)],
          sem,
      ).wait()

    issue(0)

    def body(g, carry):
      @pl.when(g + 1 < n_hgroups)
      def _():
        issue(g + 1)

      drain(g)
      h = (g % 2) * half
      lvec = lax.bitwise_and(dup_idx(g), BLK - 1)
      cols = h + k8 * BLK + lvec
      ocols = g * HG + k8 + sel * b_per_w
      for d in range(hd):
        rows = d + sel * hd
        vals = plsc.load_gather(ring, [rows, cols])
        plsc.store_scatter(out_stage, [jnp.full((L,), d, jnp.int32), ocols], vals)
      return carry

    lax.fori_loop(0, n_hgroups, body, 0)
    pltpu.sync_copy(
        out_stage.at[:, pl.ds(0, b_per_w)],
        out_hbm.at[pl.ds(0, hd), pl.ds(base, b_per_w)],
    )
    pltpu.sync_copy(
        out_stage.at[:, pl.ds(b_per_w, b_per_w)],
        out_hbm.at[pl.ds(hd, hd), pl.ds(base, b_per_w)],
    )

  return run


def kernel(idx, emb_weight):
  B = idx.shape[0]
  V, D = emb_weight.shape
  run = _lookup_kernel(B, V, D)
  out_t = run(idx.astype(jnp.int32), emb_weight.T)
  return out_t.T
